# Initial kernel scaffold; baseline (speedup 1.0000x reference)
import jax
import jax.numpy as jnp
from jax.experimental import pallas as pl


def kernel(xyz, new_xyz, features):
    raise NotImplementedError("write your pallas kernel here")


# speedup vs baseline: 4.4375x; 4.4375x over previous
import functools

import jax
import jax.numpy as jnp
from jax import lax
from jax.experimental import pallas as pl
from jax.experimental.pallas import tpu as pltpu
from jax.experimental.pallas import tpu_sc as plsc

_RADIUS = 0.1
_K = 32
_NCORES = 2
_NSUB = 16
_NWORKERS = _NCORES * _NSUB
_LANES = 16


def _worker_id():
    return lax.axis_index("s") * _NCORES + lax.axis_index("c")


def _splat_i32(x):
    return jnp.full((_LANES,), x, jnp.int32)


def _ball_body(N, S, s_per_w, xyzt, ctrt, idx_out, xv, yv, zv, cxv, cyv, czv,
               sbuf, iout):
    w = _worker_id()
    per_b = S // s_per_w
    b = w // per_b
    s0 = (w % per_b) * s_per_w

    pltpu.sync_copy(xyzt.at[b, 0], xv)
    pltpu.sync_copy(xyzt.at[b, 1], yv)
    pltpu.sync_copy(xyzt.at[b, 2], zv)
    pltpu.sync_copy(ctrt.at[b, 0, pl.ds(s0, s_per_w)], cxv)
    pltpu.sync_copy(ctrt.at[b, 1, pl.ds(s0, s_per_w)], cyv)
    pltpu.sync_copy(ctrt.at[b, 2, pl.ds(s0, s_per_w)], czv)

    iota = lax.iota(jnp.int32, _LANES)
    zeros_i = jnp.zeros((_LANES,), jnp.int32)
    r2 = jnp.float32(_RADIUS * _RADIUS)
    nchunks = N // _LANES

    def centroid_body(i, carry):
        isp = _splat_i32(i)
        cx = plsc.load_gather(cxv, [isp])
        cy = plsc.load_gather(cyv, [isp])
        cz = plsc.load_gather(czv, [isp])
        sbuf[pl.ds(0, _LANES)] = zeros_i

        def chunk(j, fv):
            base = j * _LANES
            dx = xv[pl.ds(base, _LANES)] - cx
            dy = yv[pl.ds(base, _LANES)] - cy
            dz = zv[pl.ds(base, _LANES)] - cz
            d2 = dx * dx + dy * dy + dz * dz
            m = d2 < r2

            def hit(f):
                cs = plsc.cumsum(m.astype(jnp.int32))
                pos = jnp.minimum(f + cs - 1, 47)
                plsc.store_scatter(sbuf, [pos], iota + base, mask=m)
                return f + plsc.all_reduce_population_count(m)

            return lax.cond(jnp.any(m), hit, lambda f: f, fv)

        found = lax.fori_loop(0, nchunks, chunk, zeros_i)
        first = plsc.load_gather(sbuf, [jnp.minimum(found, 0)])
        for h in range(_K // _LANES):
            cur = sbuf[pl.ds(h * _LANES, _LANES)]
            posv = iota + h * _LANES
            iout[i, pl.ds(h * _LANES, _LANES)] = jnp.where(
                posv < found, cur, first)
        return carry

    lax.fori_loop(0, s_per_w, centroid_body, 0)
    pltpu.sync_copy(iout, idx_out.at[b, pl.ds(s0, s_per_w), :])


def _gather_body(N, S, CH, rows, ctrt, idx_in, out, idxv, rowv, ctrv, outv):
    w = _worker_id()
    per_b = _NWORKERS // rows.shape[0]
    b = w // per_b
    g = w % per_b

    pltpu.sync_copy(idx_in.at[b], idxv)
    pltpu.sync_copy(ctrt.at[b], ctrv)

    nchan = (CH - g + per_b - 1) // per_b
    zeros_f = jnp.zeros((_LANES,), jnp.float32)

    def chan_body(ci, carry):
        c = g + ci * per_b
        pltpu.sync_copy(rows.at[b, c], rowv)
        csafe = jnp.minimum(c, 2)
        is_xyz = jnp.broadcast_to(c < 3, (_LANES,))

        def s_body(s, carry2):
            sub = plsc.load_gather(ctrv, [_splat_i32(csafe), _splat_i32(s)])
            sub = jnp.where(is_xyz, sub, zeros_f)
            for h in range(_K // _LANES):
                ids = idxv[s, pl.ds(h * _LANES, _LANES)]
                vals = plsc.load_gather(rowv, [ids])
                outv[s, pl.ds(h * _LANES, _LANES)] = vals - sub
            return carry2

        lax.fori_loop(0, S, s_body, 0)
        pltpu.sync_copy(outv, out.at[b, c])
        return carry

    lax.fori_loop(0, nchan, chan_body, 0)


@jax.jit
def kernel(xyz, new_xyz, features):
    B, N, _ = xyz.shape
    S = new_xyz.shape[1]
    C = features.shape[1]
    CH = C + 3
    s_per_w = S // (_NWORKERS // B)

    xyzt = jnp.transpose(xyz, (0, 2, 1))
    ctrt = jnp.transpose(new_xyz, (0, 2, 1))
    rows = jnp.concatenate([xyzt, features], axis=1)

    mesh = plsc.VectorSubcoreMesh(core_axis_name="c", subcore_axis_name="s")
    cparams = pltpu.CompilerParams(
        use_tc_tiling_on_sc=False, needs_layout_passes=False)

    ball = pl.kernel(
        functools.partial(_ball_body, N, S, s_per_w),
        out_type=jax.ShapeDtypeStruct((B, S, _K), jnp.int32),
        mesh=mesh,
        scratch_types=[
            pltpu.VMEM((N,), jnp.float32),
            pltpu.VMEM((N,), jnp.float32),
            pltpu.VMEM((N,), jnp.float32),
            pltpu.VMEM((s_per_w,), jnp.float32),
            pltpu.VMEM((s_per_w,), jnp.float32),
            pltpu.VMEM((s_per_w,), jnp.float32),
            pltpu.VMEM((48,), jnp.int32),
            pltpu.VMEM((s_per_w, _K), jnp.int32),
        ],
        compiler_params=cparams,
    )
    idx = ball(xyzt, ctrt)

    gather = pl.kernel(
        functools.partial(_gather_body, N, S, CH),
        out_type=jax.ShapeDtypeStruct((B, CH, S, _K), jnp.float32),
        mesh=mesh,
        scratch_types=[
            pltpu.VMEM((S, _K), jnp.int32),
            pltpu.VMEM((N,), jnp.float32),
            pltpu.VMEM((3, S), jnp.float32),
            pltpu.VMEM((S, _K), jnp.float32),
        ],
        compiler_params=cparams,
    )
    return gather(rows, ctrt, idx)


# speedup vs baseline: 14.9169x; 3.3616x over previous
import functools

import jax
import jax.numpy as jnp
from jax import lax
from jax.experimental import pallas as pl
from jax.experimental.pallas import tpu as pltpu
from jax.experimental.pallas import tpu_sc as plsc

_RADIUS = 0.1
_K = 32
_NCORES = 2
_NSUB = 16
_NWORKERS = _NCORES * _NSUB
_LANES = 16


def _worker_id():
    return lax.axis_index("s") * _NCORES + lax.axis_index("c")


def _splat_i32(x):
    return jnp.full((_LANES,), x, jnp.int32)


def _ball_body(N, S, s_per_w, xyzt, ctrt, idx_out, xv, yv, zv, cxv, cyv, czv,
               sbuf, iout):
    w = _worker_id()
    per_b = S // s_per_w
    b = w // per_b
    s0 = (w % per_b) * s_per_w

    pltpu.sync_copy(xyzt.at[b, 0], xv)
    pltpu.sync_copy(xyzt.at[b, 1], yv)
    pltpu.sync_copy(xyzt.at[b, 2], zv)
    pltpu.sync_copy(ctrt.at[b, 0, pl.ds(s0, s_per_w)], cxv)
    pltpu.sync_copy(ctrt.at[b, 1, pl.ds(s0, s_per_w)], cyv)
    pltpu.sync_copy(ctrt.at[b, 2, pl.ds(s0, s_per_w)], czv)

    iota = lax.iota(jnp.int32, _LANES)
    zeros_i = jnp.zeros((_LANES,), jnp.int32)
    r2 = jnp.float32(_RADIUS * _RADIUS)
    nchunks = N // _LANES

    def centroid_body(i, carry):
        isp = _splat_i32(i)
        cx = plsc.load_gather(cxv, [isp])
        cy = plsc.load_gather(cyv, [isp])
        cz = plsc.load_gather(czv, [isp])
        sbuf[pl.ds(0, _LANES)] = zeros_i

        def chunk(j, fv):
            base = j * _LANES
            dx = xv[pl.ds(base, _LANES)] - cx
            dy = yv[pl.ds(base, _LANES)] - cy
            dz = zv[pl.ds(base, _LANES)] - cz
            d2 = dx * dx + dy * dy + dz * dz
            m = d2 < r2
            cs = plsc.cumsum(m.astype(jnp.int32))
            pos = jnp.minimum(fv + cs - 1, 47)
            plsc.store_scatter(sbuf, [pos], iota + base, mask=m)
            return fv + plsc.all_reduce_population_count(m)

        found = plsc.parallel_loop(0, nchunks, carry=zeros_i, unroll=4)(chunk)
        first = plsc.load_gather(sbuf, [jnp.minimum(found, 0)])
        for h in range(_K // _LANES):
            cur = sbuf[pl.ds(h * _LANES, _LANES)]
            posv = iota + h * _LANES
            iout[i, pl.ds(h * _LANES, _LANES)] = jnp.where(
                posv < found, cur, first)
        return carry

    lax.fori_loop(0, s_per_w, centroid_body, 0)
    pltpu.sync_copy(iout, idx_out.at[b, pl.ds(s0, s_per_w), :])


def _gather_body(N, S, CH, rows, ctrt, idx_in, out, idxv, row0, row1, ctrv,
                 out0, out1, lsem, ssem0, ssem1):
    w = _worker_id()
    per_b = _NWORKERS // rows.shape[0]
    b = w // per_b
    g = w % per_b
    nfeat = (CH - 3) // per_b

    pltpu.sync_copy(idx_in.at[b], idxv)
    rowbufs = [row0, row1]
    outbufs = [out0, out1]

    pltpu.async_copy(rows.at[b, 3 + g], row0, lsem).wait()

    def gather_into(rowv, outv):
        def s_body(s, carry):
            for h in range(_K // _LANES):
                ids = idxv[s, pl.ds(h * _LANES, _LANES)]
                outv[s, pl.ds(h * _LANES, _LANES)] = plsc.load_gather(
                    rowv, [ids])
            return carry
        lax.fori_loop(0, S, s_body, 0, unroll=4)

    osem = [ssem0, ssem1]
    for ci in range(nfeat):
        cur = ci % 2
        nxt = 1 - cur
        c = 3 + g + ci * per_b
        if ci + 1 < nfeat:
            ncopy = pltpu.make_async_copy(
                rows.at[b, c + per_b], rowbufs[nxt], lsem)
            ncopy.start()
        if ci >= 2:
            pltpu.make_async_copy(
                outbufs[cur], out.at[b, c], osem[cur]).wait()
        gather_into(rowbufs[cur], outbufs[cur])
        pltpu.make_async_copy(outbufs[cur], out.at[b, c], osem[cur]).start()
        if ci + 1 < nfeat:
            ncopy.wait()
    pltpu.make_async_copy(out0, out.at[b, 3 + g], ssem0).wait()
    pltpu.make_async_copy(out1, out.at[b, 3 + g], ssem1).wait()

    @pl.when(g < 3)
    def _():
        pltpu.sync_copy(rows.at[b, g], row0)
        pltpu.sync_copy(ctrt.at[b, jnp.minimum(g, 2)], ctrv)

        def s_body(s, carry):
            sub = plsc.load_gather(ctrv, [_splat_i32(s)])
            for h in range(_K // _LANES):
                ids = idxv[s, pl.ds(h * _LANES, _LANES)]
                vals = plsc.load_gather(row0, [ids])
                out0[s, pl.ds(h * _LANES, _LANES)] = vals - sub
            return carry

        lax.fori_loop(0, S, s_body, 0, unroll=4)
        pltpu.sync_copy(out0, out.at[b, g])


@jax.jit
def kernel(xyz, new_xyz, features):
    B, N, _ = xyz.shape
    S = new_xyz.shape[1]
    C = features.shape[1]
    CH = C + 3
    s_per_w = S // (_NWORKERS // B)

    xyzt = jnp.transpose(xyz, (0, 2, 1))
    ctrt = jnp.transpose(new_xyz, (0, 2, 1))
    rows = jnp.concatenate([xyzt, features], axis=1)

    mesh = plsc.VectorSubcoreMesh(core_axis_name="c", subcore_axis_name="s")
    cparams = pltpu.CompilerParams(
        use_tc_tiling_on_sc=False, needs_layout_passes=False)

    ball = pl.kernel(
        functools.partial(_ball_body, N, S, s_per_w),
        out_type=jax.ShapeDtypeStruct((B, S, _K), jnp.int32),
        mesh=mesh,
        scratch_types=[
            pltpu.VMEM((N,), jnp.float32),
            pltpu.VMEM((N,), jnp.float32),
            pltpu.VMEM((N,), jnp.float32),
            pltpu.VMEM((s_per_w,), jnp.float32),
            pltpu.VMEM((s_per_w,), jnp.float32),
            pltpu.VMEM((s_per_w,), jnp.float32),
            pltpu.VMEM((48,), jnp.int32),
            pltpu.VMEM((s_per_w, _K), jnp.int32),
        ],
        compiler_params=cparams,
    )
    idx = ball(xyzt, ctrt)

    gather = pl.kernel(
        functools.partial(_gather_body, N, S, CH),
        out_type=jax.ShapeDtypeStruct((B, CH, S, _K), jnp.float32),
        mesh=mesh,
        scratch_types=[
            pltpu.VMEM((S, _K), jnp.int32),
            pltpu.VMEM((N,), jnp.float32),
            pltpu.VMEM((N,), jnp.float32),
            pltpu.VMEM((S,), jnp.float32),
            pltpu.VMEM((S, _K), jnp.float32),
            pltpu.VMEM((S, _K), jnp.float32),
            pltpu.SemaphoreType.DMA,
            pltpu.SemaphoreType.DMA,
            pltpu.SemaphoreType.DMA,
        ],
        compiler_params=cparams,
    )
    return gather(rows, ctrt, idx)


# speedup vs baseline: 20.9917x; 1.4072x over previous
import functools

import jax
import jax.numpy as jnp
from jax import lax
from jax.experimental import pallas as pl
from jax.experimental.pallas import tpu as pltpu
from jax.experimental.pallas import tpu_sc as plsc

_RADIUS = 0.1
_K = 32
_NCORES = 2
_NSUB = 16
_NWORKERS = _NCORES * _NSUB
_LANES = 16


def _worker_id():
    return lax.axis_index("s") * _NCORES + lax.axis_index("c")


def _splat_i32(x):
    return jnp.full((_LANES,), x, jnp.int32)


def _ball_body(N, S, s_per_w, xyzt, ctrt, idx_out, xv, yv, zv, cxv, cyv, czv,
               sbuf, iout):
    w = _worker_id()
    per_b = S // s_per_w
    b = w // per_b
    s0 = (w % per_b) * s_per_w

    pltpu.sync_copy(xyzt.at[b, 0], xv)
    pltpu.sync_copy(xyzt.at[b, 1], yv)
    pltpu.sync_copy(xyzt.at[b, 2], zv)
    pltpu.sync_copy(ctrt.at[b, 0, pl.ds(s0, s_per_w)], cxv)
    pltpu.sync_copy(ctrt.at[b, 1, pl.ds(s0, s_per_w)], cyv)
    pltpu.sync_copy(ctrt.at[b, 2, pl.ds(s0, s_per_w)], czv)

    iota = lax.iota(jnp.int32, _LANES)
    zeros_i = jnp.zeros((_LANES,), jnp.int32)
    r2 = jnp.float32(_RADIUS * _RADIUS)
    nchunks = N // _LANES

    def centroid_body(i, carry):
        isp = _splat_i32(i)
        cx = plsc.load_gather(cxv, [isp])
        cy = plsc.load_gather(cyv, [isp])
        cz = plsc.load_gather(czv, [isp])
        sbuf[pl.ds(0, _LANES)] = zeros_i

        def chunk(j, fv):
            base = j * _LANES
            dx = xv[pl.ds(base, _LANES)] - cx
            dy = yv[pl.ds(base, _LANES)] - cy
            dz = zv[pl.ds(base, _LANES)] - cz
            d2 = dx * dx + dy * dy + dz * dz
            m = d2 < r2
            cs = plsc.cumsum(m.astype(jnp.int32))
            pos = jnp.minimum(fv + cs - 1, 47)
            plsc.store_scatter(sbuf, [pos], iota + base, mask=m)
            return fv + plsc.all_reduce_population_count(m)

        found = plsc.parallel_loop(0, nchunks, carry=zeros_i, unroll=4)(chunk)
        first = plsc.load_gather(sbuf, [jnp.minimum(found, 0)])
        for h in range(_K // _LANES):
            cur = sbuf[pl.ds(h * _LANES, _LANES)]
            posv = iota + h * _LANES
            iout[i, pl.ds(h * _LANES, _LANES)] = jnp.where(
                posv < found, cur, first)
        return carry

    lax.fori_loop(0, s_per_w, centroid_body, 0)
    pltpu.sync_copy(iout, idx_out.at[b, pl.ds(s0, s_per_w), :])


def _gather_body(N, S, CH, xyzt, feats, ctrt, idx_in, out, idxv, row0, row1,
                 ctrv, out0, out1, lsem, ssem0, ssem1):
    w = _worker_id()
    per_b = _NWORKERS // feats.shape[0]
    b = w // per_b
    g = w % per_b
    nfeat = (CH - 3) // per_b

    pltpu.sync_copy(idx_in.at[b], idxv)
    rowbufs = [row0, row1]
    outbufs = [out0, out1]

    pltpu.async_copy(feats.at[b, g], row0, lsem).wait()

    def gather_into(rowv, outv):
        def s_body(s):
            for h in range(_K // _LANES):
                ids = idxv[s, pl.ds(h * _LANES, _LANES)]
                outv[s, pl.ds(h * _LANES, _LANES)] = plsc.load_gather(
                    rowv, [ids])
        plsc.parallel_loop(0, S, unroll=4)(s_body)

    osem = [ssem0, ssem1]
    for ci in range(nfeat):
        cur = ci % 2
        nxt = 1 - cur
        c = 3 + g + ci * per_b
        if ci + 1 < nfeat:
            ncopy = pltpu.make_async_copy(
                feats.at[b, c + per_b - 3], rowbufs[nxt], lsem)
            ncopy.start()
        if ci >= 2:
            pltpu.make_async_copy(
                outbufs[cur], out.at[b, c], osem[cur]).wait()
        gather_into(rowbufs[cur], outbufs[cur])
        pltpu.make_async_copy(outbufs[cur], out.at[b, c], osem[cur]).start()
        if ci + 1 < nfeat:
            ncopy.wait()
    pltpu.make_async_copy(out0, out.at[b, 3 + g], ssem0).wait()
    pltpu.make_async_copy(out1, out.at[b, 3 + g], ssem1).wait()

    @pl.when(g < 3)
    def _():
        pltpu.sync_copy(xyzt.at[b, jnp.minimum(g, 2)], row0)
        pltpu.sync_copy(ctrt.at[b, jnp.minimum(g, 2)], ctrv)

        def s_body(s):
            sub = plsc.load_gather(ctrv, [_splat_i32(s)])
            for h in range(_K // _LANES):
                ids = idxv[s, pl.ds(h * _LANES, _LANES)]
                vals = plsc.load_gather(row0, [ids])
                out0[s, pl.ds(h * _LANES, _LANES)] = vals - sub

        plsc.parallel_loop(0, S, unroll=4)(s_body)
        pltpu.sync_copy(out0, out.at[b, g])


@jax.jit
def kernel(xyz, new_xyz, features):
    B, N, _ = xyz.shape
    S = new_xyz.shape[1]
    C = features.shape[1]
    CH = C + 3
    s_per_w = S // (_NWORKERS // B)

    xyzt = jnp.transpose(xyz, (0, 2, 1))
    ctrt = jnp.transpose(new_xyz, (0, 2, 1))

    mesh = plsc.VectorSubcoreMesh(core_axis_name="c", subcore_axis_name="s")
    cparams = pltpu.CompilerParams(
        use_tc_tiling_on_sc=False, needs_layout_passes=False)

    ball = pl.kernel(
        functools.partial(_ball_body, N, S, s_per_w),
        out_type=jax.ShapeDtypeStruct((B, S, _K), jnp.int32),
        mesh=mesh,
        scratch_types=[
            pltpu.VMEM((N,), jnp.float32),
            pltpu.VMEM((N,), jnp.float32),
            pltpu.VMEM((N,), jnp.float32),
            pltpu.VMEM((s_per_w,), jnp.float32),
            pltpu.VMEM((s_per_w,), jnp.float32),
            pltpu.VMEM((s_per_w,), jnp.float32),
            pltpu.VMEM((48,), jnp.int32),
            pltpu.VMEM((s_per_w, _K), jnp.int32),
        ],
        compiler_params=cparams,
    )
    idx = ball(xyzt, ctrt)

    gather = pl.kernel(
        functools.partial(_gather_body, N, S, CH),
        out_type=jax.ShapeDtypeStruct((B, CH, S, _K), jnp.float32),
        mesh=mesh,
        scratch_types=[
            pltpu.VMEM((S, _K), jnp.int32),
            pltpu.VMEM((N,), jnp.float32),
            pltpu.VMEM((N,), jnp.float32),
            pltpu.VMEM((S,), jnp.float32),
            pltpu.VMEM((S, _K), jnp.float32),
            pltpu.VMEM((S, _K), jnp.float32),
            pltpu.SemaphoreType.DMA,
            pltpu.SemaphoreType.DMA,
            pltpu.SemaphoreType.DMA,
        ],
        compiler_params=cparams,
    )
    return gather(xyzt, features, ctrt, idx)
